# baseline (speedup 1.0000x reference)
import functools

import jax
import jax.numpy as jnp
from jax import lax
from jax.experimental import pallas as pl
from jax.experimental.pallas import tpu as pltpu
from jax.experimental.pallas import tpu_sc as plsc

N_NODES = 10000
N_EDGES = 320000
D = 128
D_OUT = 64

NP = 10240
EP = 327680
PAD_NODE = 10100

NW = 32
EDGES_PER_TILE = EP // NW
K = 32
SB = 8
AL_R = 80
CHUNKS = EDGES_PER_TILE // K
ROWS_PER_TILE = NP // 16
ZR = 128

_R = 2048
_G = NP // _R
_AR = _R // D


def _tc_first_body(x_ref, ws_ref, wd_ref, as_ref, ad_ref, hs_ref, als_ref, ald_ref):
    x = x_ref[...]
    hs = jnp.dot(x, ws_ref[...], preferred_element_type=jnp.float32)
    hs_ref[...] = hs
    als = jnp.dot(hs, as_ref[...], preferred_element_type=jnp.float32)
    als_ref[...] = als.reshape(_AR, D)
    v = jnp.dot(wd_ref[...], ad_ref[...], preferred_element_type=jnp.float32)
    ald_ref[...] = jnp.dot(x, v, preferred_element_type=jnp.float32).reshape(_AR, D)


def _tc_mid_body(acc_ref, s_ref, b_ref, w_ref, as_ref, ad_ref, hs_ref, als_ref, ald_ref):
    num = acc_ref[0] + acc_ref[1]
    s = jnp.sum(s_ref[...], axis=1, keepdims=True)
    h = jnp.maximum(jnp.where(s > 0.0, num / s, 0.0) + b_ref[...], 0.0)
    hs = jnp.dot(h, w_ref[...], preferred_element_type=jnp.float32)
    hs_ref[...] = hs
    als = jnp.dot(hs, as_ref[...], preferred_element_type=jnp.float32)
    als_ref[...] = als.reshape(_AR, D)
    v = jnp.dot(w_ref[...], ad_ref[...], preferred_element_type=jnp.float32)
    ald_ref[...] = jnp.dot(h, v, preferred_element_type=jnp.float32).reshape(_AR, D)


def _tc_last_body(acc_ref, s_ref, b_ref, w1_ref, b1_ref, w2_ref, b2_ref, out_ref):
    num = acc_ref[0] + acc_ref[1]
    s = jnp.sum(s_ref[...], axis=1, keepdims=True)
    h = jnp.maximum(jnp.where(s > 0.0, num / s, 0.0) + b_ref[...], 0.0)
    h = jnp.maximum(jnp.dot(h, w1_ref[...], preferred_element_type=jnp.float32)
                    + b1_ref[...], 0.0)
    out_ref[...] = jnp.dot(h, w2_ref[...], preferred_element_type=jnp.float32) + b2_ref[...]


def _row_blk(i):
    return (i, 0)


def _acc_blk(i):
    return (0, i, 0)


def _full_blk(i):
    return (0, 0)


_W_SPEC = pl.BlockSpec((D, D), _full_blk)
_A_SPEC = pl.BlockSpec((D, 1), _full_blk)
_B_SPEC = pl.BlockSpec((1, D), _full_blk)
_H_SPEC = pl.BlockSpec((_R, D), _row_blk)
_AL_SPEC = pl.BlockSpec((_AR, D), _row_blk)
_ACC_SPEC = pl.BlockSpec((2, _R, D), _acc_blk)
_S_SPEC = pl.BlockSpec((_R, NW), _row_blk)

_PROJ_OUT = (jax.ShapeDtypeStruct((NP, D), jnp.float32),
             jax.ShapeDtypeStruct((NP // D, D), jnp.float32),
             jax.ShapeDtypeStruct((NP // D, D), jnp.float32))

_tc_first = pl.pallas_call(
    _tc_first_body, grid=(_G,),
    in_specs=[_H_SPEC, _W_SPEC, _W_SPEC, _A_SPEC, _A_SPEC],
    out_specs=[_H_SPEC, _AL_SPEC, _AL_SPEC],
    out_shape=_PROJ_OUT)

_tc_mid = pl.pallas_call(
    _tc_mid_body, grid=(_G,),
    in_specs=[_ACC_SPEC, _S_SPEC, _B_SPEC, _W_SPEC, _A_SPEC, _A_SPEC],
    out_specs=[_H_SPEC, _AL_SPEC, _AL_SPEC],
    out_shape=_PROJ_OUT)

_tc_last = pl.pallas_call(
    _tc_last_body, grid=(_G,),
    in_specs=[_ACC_SPEC, _S_SPEC, _B_SPEC, _W_SPEC, _B_SPEC, _W_SPEC, _B_SPEC],
    out_specs=_H_SPEC,
    out_shape=jax.ShapeDtypeStruct((NP, D), jnp.float32))


def _sc_edge_body(hs_hbm, src_hbm, dst_hbm, as_hbm, ad_hbm, out_hbm, s_hbm,
                  src_sb, dst_sb, as_v, ad_v, ee_v, rows_v, s_loc, acc_sp,
                  gsem, ssem, isem):
    cid = lax.axis_index("c")
    sid = lax.axis_index("s")
    wid = sid * 2 + cid

    pltpu.sync_copy(as_hbm.at[pl.ds(0, AL_R)], as_v)
    pltpu.sync_copy(ad_hbm.at[pl.ds(0, AL_R)], ad_v)

    zero16 = jnp.zeros((16,), jnp.float32)

    def _z(r, carry):
        for c in range(D // 16):
            rows_v[0, r, pl.ds(c * 16, 16)] = zero16
        return carry
    lax.fori_loop(0, K, _z, 0)

    def _zs(r, carry):
        for c in range(128 // 16):
            s_loc[r, pl.ds(c * 16, 16)] = zero16
        return carry
    lax.fori_loop(0, AL_R, _zs, 0)

    def _za(t, carry):
        pltpu.async_copy(rows_v.at[0],
                         acc_sp.at[pl.ds(sid * ROWS_PER_TILE + t * K, K)], gsem)
        return carry
    lax.fori_loop(0, ROWS_PER_TILE // K, _za, 0)

    def _zw(t, carry):
        pltpu.make_async_copy(
            rows_v.at[0],
            acc_sp.at[pl.ds(sid * ROWS_PER_TILE + t * K, K)], gsem).wait()
        return carry
    lax.fori_loop(0, ROWS_PER_TILE // K, _zw, 0)
    plsc.subcore_barrier()

    NSB = CHUNKS // SB

    def _stage_idx(sb, islot):
        base = wid * CHUNKS + sb * SB
        pltpu.async_copy(src_hbm.at[pl.ds(base, SB)], src_sb.at[islot], isem)
        pltpu.async_copy(dst_hbm.at[pl.ds(base, SB)], dst_sb.at[islot], isem)

    def _wait_idx(sb, islot):
        base = wid * CHUNKS + sb * SB
        pltpu.make_async_copy(src_hbm.at[pl.ds(base, SB)], src_sb.at[islot],
                              isem).wait()
        pltpu.make_async_copy(dst_hbm.at[pl.ds(base, SB)], dst_sb.at[islot],
                              isem).wait()

    _stage_idx(0, 0)

    def _sblock(sb, carry):
        islot = sb % 2
        _wait_idx(sb, islot)

        @pl.when(sb < NSB - 1)
        def _():
            _stage_idx(sb + 1, 1 - islot)

        pltpu.async_copy(hs_hbm.at[src_sb.at[islot, 0]], rows_v.at[0], gsem)

        def _chunk(j, c1):
            b = j % 2
            def _ee(t, c2):
                sv = src_sb[islot, j, pl.ds(t * 16, 16)]
                dv = dst_sb[islot, j, pl.ds(t * 16, 16)]
                e = (plsc.load_gather(as_v, [sv >> 7, sv & 127])
                     + plsc.load_gather(ad_v, [dv >> 7, dv & 127]))
                e = jnp.where(e > 0.0, e, 0.2 * e)
                ee = jnp.exp(e)
                ee_v[t] = ee
                plsc.addupdate_scatter(s_loc, [dv >> 7, dv & 127], ee)
                return c2
            lax.fori_loop(0, K // 16, _ee, 0)

            pltpu.make_async_copy(hs_hbm.at[src_sb.at[islot, j]],
                                  rows_v.at[b], gsem).wait()

            @pl.when(j >= 1)
            def _():
                pltpu.make_async_copy(rows_v.at[1 - b],
                                      acc_sp.at[dst_sb.at[islot, j - 1]],
                                      ssem).wait()

            @pl.when(j < SB - 1)
            def _():
                pltpu.async_copy(hs_hbm.at[src_sb.at[islot, j + 1]],
                                 rows_v.at[1 - b], gsem)

            def _row(i):
                w = plsc.load_gather(
                    ee_v, [jnp.broadcast_to(i >> 4, (16,)),
                           jnp.broadcast_to(i & 15, (16,))])
                for c in range(D // 16):
                    rows_v[b, i, pl.ds(c * 16, 16)] = (
                        rows_v[b, i, pl.ds(c * 16, 16)] * w)
            plsc.parallel_loop(0, K, 1, unroll=4)(_row)

            pltpu.async_copy(rows_v.at[b], acc_sp.at[dst_sb.at[islot, j]],
                             ssem, add=True)
            return c1
        lax.fori_loop(0, SB, _chunk, 0)

        pltpu.make_async_copy(rows_v.at[(SB - 1) % 2],
                              acc_sp.at[dst_sb.at[islot, SB - 1]], ssem).wait()
        return carry
    lax.fori_loop(0, NSB, _sblock, 0)

    pltpu.sync_copy(s_loc, s_hbm.at[wid])
    plsc.subcore_barrier()

    def _wb(t, carry):
        pltpu.sync_copy(
            acc_sp.at[pl.ds(sid * ROWS_PER_TILE + t * ZR, ZR)],
            out_hbm.at[cid, pl.ds(sid * ROWS_PER_TILE + t * ZR, ZR)])
        return carry
    lax.fori_loop(0, ROWS_PER_TILE // ZR, _wb, 0)


_sc_edge = functools.partial(
    pl.kernel,
    out_type=(jax.ShapeDtypeStruct((2, NP, D), jnp.float32),
              jax.ShapeDtypeStruct((NW, AL_R, 128), jnp.float32)),
    mesh=plsc.VectorSubcoreMesh(core_axis_name="c", subcore_axis_name="s"),
    scratch_types=[
        pltpu.VMEM((2, SB, K), jnp.int32),
        pltpu.VMEM((2, SB, K), jnp.int32),
        pltpu.VMEM((AL_R, 128), jnp.float32),
        pltpu.VMEM((AL_R, 128), jnp.float32),
        pltpu.VMEM((K // 16, 16), jnp.float32),
        pltpu.VMEM((2, K, D), jnp.float32),
        pltpu.VMEM((AL_R, 128), jnp.float32),
        pltpu.VMEM_SHARED((NP, D), jnp.float32),
        pltpu.SemaphoreType.DMA,
        pltpu.SemaphoreType.DMA,
        pltpu.SemaphoreType.DMA,
    ],
    compiler_params=pltpu.CompilerParams(needs_layout_passes=False),
    )(_sc_edge_body)


def kernel(x, edge_index, W1s, W1d, a1s, a1d, b1, W2, a2s, a2d, b2,
           W3, a3s, a3d, b3, lin1_W, lin1_b, lin2_W, lin2_b):
    f32 = jnp.float32
    x_p = jnp.zeros((NP, D), f32).at[:N_NODES].set(x)
    pad = jnp.full((EP - N_EDGES,), PAD_NODE, jnp.int32)
    src = jnp.concatenate([edge_index[0], pad]).reshape(EP // K, K)
    dst = jnp.concatenate([edge_index[1], pad]).reshape(EP // K, K)

    def col(a):
        return a.reshape(D, 1)

    def row(a, w=D):
        return a.reshape(1, w)

    def tr(sv):
        svt = sv.reshape(NW, AL_R * 128).transpose(1, 0)
        return jnp.zeros((NP, NW), jnp.float32).at[:AL_R * 128].set(svt)

    hs, als, ald = _tc_first(x_p, W1s, W1d, col(a1s), col(a1d))
    acc, sv = _sc_edge(hs, src, dst, als, ald)
    hs, als, ald = _tc_mid(acc, tr(sv), row(b1), W2, col(a2s), col(a2d))
    acc, sv = _sc_edge(hs, src, dst, als, ald)
    hs, als, ald = _tc_mid(acc, tr(sv), row(b2), W3, col(a3s), col(a3d))
    acc, sv = _sc_edge(hs, src, dst, als, ald)

    w2p = jnp.zeros((D, D), f32).at[:, :D_OUT].set(lin2_W)
    b2p = jnp.zeros((D,), f32).at[:D_OUT].set(lin2_b)
    out = _tc_last(acc, tr(sv), row(b3), lin1_W, row(lin1_b), w2p, row(b2p))
    return out[:N_NODES, :D_OUT]


# speedup vs baseline: 1.1748x; 1.1748x over previous
import functools

import jax
import jax.numpy as jnp
from jax import lax
from jax.experimental import pallas as pl
from jax.experimental.pallas import tpu as pltpu
from jax.experimental.pallas import tpu_sc as plsc

N_NODES = 10000
N_EDGES = 320000
D = 128
D_OUT = 64

NP = 10240
EP = 327680
PAD_NODE = 10100

NW = 32
EDGES_PER_TILE = EP // NW
K = 64
SB = 4
AL_R = 80
CHUNKS = EDGES_PER_TILE // K
ROWS_PER_TILE = NP // 16
ZR = 128

_R = 2048
_G = NP // _R
_AR = _R // D


def _tc_first_body(x_ref, ws_ref, wd_ref, as_ref, ad_ref, hs_ref, als_ref, ald_ref):
    x = x_ref[...]
    hs = jnp.dot(x, ws_ref[...], preferred_element_type=jnp.float32)
    hs_ref[...] = hs
    als = jnp.dot(hs, as_ref[...], preferred_element_type=jnp.float32)
    als_ref[...] = als.reshape(_AR, D)
    v = jnp.dot(wd_ref[...], ad_ref[...], preferred_element_type=jnp.float32)
    ald_ref[...] = jnp.dot(x, v, preferred_element_type=jnp.float32).reshape(_AR, D)


def _tc_mid_body(acc_ref, s_ref, b_ref, w_ref, as_ref, ad_ref, hs_ref, als_ref, ald_ref):
    num = acc_ref[0] + acc_ref[1]
    s = jnp.sum(s_ref[...], axis=1, keepdims=True)
    h = jnp.maximum(jnp.where(s > 0.0, num / s, 0.0) + b_ref[...], 0.0)
    hs = jnp.dot(h, w_ref[...], preferred_element_type=jnp.float32)
    hs_ref[...] = hs
    als = jnp.dot(hs, as_ref[...], preferred_element_type=jnp.float32)
    als_ref[...] = als.reshape(_AR, D)
    v = jnp.dot(w_ref[...], ad_ref[...], preferred_element_type=jnp.float32)
    ald_ref[...] = jnp.dot(h, v, preferred_element_type=jnp.float32).reshape(_AR, D)


def _tc_last_body(acc_ref, s_ref, b_ref, w1_ref, b1_ref, w2_ref, b2_ref, out_ref):
    num = acc_ref[0] + acc_ref[1]
    s = jnp.sum(s_ref[...], axis=1, keepdims=True)
    h = jnp.maximum(jnp.where(s > 0.0, num / s, 0.0) + b_ref[...], 0.0)
    h = jnp.maximum(jnp.dot(h, w1_ref[...], preferred_element_type=jnp.float32)
                    + b1_ref[...], 0.0)
    out_ref[...] = jnp.dot(h, w2_ref[...], preferred_element_type=jnp.float32) + b2_ref[...]


def _row_blk(i):
    return (i, 0)


def _acc_blk(i):
    return (0, i, 0)


def _full_blk(i):
    return (0, 0)


_W_SPEC = pl.BlockSpec((D, D), _full_blk)
_A_SPEC = pl.BlockSpec((D, 1), _full_blk)
_B_SPEC = pl.BlockSpec((1, D), _full_blk)
_H_SPEC = pl.BlockSpec((_R, D), _row_blk)
_AL_SPEC = pl.BlockSpec((_AR, D), _row_blk)
_ACC_SPEC = pl.BlockSpec((2, _R, D), _acc_blk)
_S_SPEC = pl.BlockSpec((_R, NW), _row_blk)

_PROJ_OUT = (jax.ShapeDtypeStruct((NP, D), jnp.float32),
             jax.ShapeDtypeStruct((NP // D, D), jnp.float32),
             jax.ShapeDtypeStruct((NP // D, D), jnp.float32))

_tc_first = pl.pallas_call(
    _tc_first_body, grid=(_G,),
    in_specs=[_H_SPEC, _W_SPEC, _W_SPEC, _A_SPEC, _A_SPEC],
    out_specs=[_H_SPEC, _AL_SPEC, _AL_SPEC],
    out_shape=_PROJ_OUT)

_tc_mid = pl.pallas_call(
    _tc_mid_body, grid=(_G,),
    in_specs=[_ACC_SPEC, _S_SPEC, _B_SPEC, _W_SPEC, _A_SPEC, _A_SPEC],
    out_specs=[_H_SPEC, _AL_SPEC, _AL_SPEC],
    out_shape=_PROJ_OUT)

_tc_last = pl.pallas_call(
    _tc_last_body, grid=(_G,),
    in_specs=[_ACC_SPEC, _S_SPEC, _B_SPEC, _W_SPEC, _B_SPEC, _W_SPEC, _B_SPEC],
    out_specs=_H_SPEC,
    out_shape=jax.ShapeDtypeStruct((NP, D), jnp.float32))


def _sc_edge_body(hs_hbm, src_hbm, dst_hbm, as_hbm, ad_hbm, out_hbm, s_hbm,
                  src_sb, dst_sb, as_v, ad_v, ee_v, rows_v, s_loc, acc_sp,
                  gsem, ssem):
    cid = lax.axis_index("c")
    sid = lax.axis_index("s")
    wid = sid * 2 + cid

    pltpu.sync_copy(as_hbm.at[pl.ds(0, AL_R)], as_v)
    pltpu.sync_copy(ad_hbm.at[pl.ds(0, AL_R)], ad_v)

    zero16 = jnp.zeros((16,), jnp.float32)

    def _z(r, carry):
        for c in range(D // 16):
            rows_v[0, r, pl.ds(c * 16, 16)] = zero16
        return carry
    lax.fori_loop(0, K, _z, 0)

    def _zs(r, carry):
        for c in range(128 // 16):
            s_loc[r, pl.ds(c * 16, 16)] = zero16
        return carry
    lax.fori_loop(0, AL_R, _zs, 0)

    def _za(t, carry):
        pltpu.async_copy(rows_v.at[0],
                         acc_sp.at[pl.ds(sid * ROWS_PER_TILE + t * K, K)], gsem)
        return carry
    lax.fori_loop(0, ROWS_PER_TILE // K, _za, 0)

    def _zw(t, carry):
        pltpu.make_async_copy(
            rows_v.at[0],
            acc_sp.at[pl.ds(sid * ROWS_PER_TILE + t * K, K)], gsem).wait()
        return carry
    lax.fori_loop(0, ROWS_PER_TILE // K, _zw, 0)
    plsc.subcore_barrier()

    NSB = CHUNKS // SB

    def _sblock(sb, carry):
        base = wid * CHUNKS + sb * SB
        pltpu.sync_copy(src_hbm.at[pl.ds(base, SB)], src_sb)
        pltpu.sync_copy(dst_hbm.at[pl.ds(base, SB)], dst_sb)

        pltpu.async_copy(hs_hbm.at[src_sb.at[0]], rows_v.at[0], gsem)

        def _chunk(j, c1):
            b = j % 2
            def _ee(t, c2):
                sv = src_sb[j, pl.ds(t * 16, 16)]
                dv = dst_sb[j, pl.ds(t * 16, 16)]
                e = (plsc.load_gather(as_v, [sv >> 7, sv & 127])
                     + plsc.load_gather(ad_v, [dv >> 7, dv & 127]))
                e = jnp.where(e > 0.0, e, 0.2 * e)
                ee = jnp.exp(e)
                ee_v[t] = ee
                plsc.addupdate_scatter(s_loc, [dv >> 7, dv & 127], ee)
                return c2
            lax.fori_loop(0, K // 16, _ee, 0)

            pltpu.make_async_copy(hs_hbm.at[src_sb.at[j]],
                                  rows_v.at[b], gsem).wait()

            @pl.when(j >= 1)
            def _():
                pltpu.make_async_copy(rows_v.at[1 - b],
                                      acc_sp.at[dst_sb.at[j - 1]],
                                      ssem).wait()

            @pl.when(j < SB - 1)
            def _():
                pltpu.async_copy(hs_hbm.at[src_sb.at[j + 1]],
                                 rows_v.at[1 - b], gsem)

            def _row(i):
                w = plsc.load_gather(
                    ee_v, [jnp.broadcast_to(i >> 4, (16,)),
                           jnp.broadcast_to(i & 15, (16,))])
                for c in range(D // 16):
                    rows_v[b, i, pl.ds(c * 16, 16)] = (
                        rows_v[b, i, pl.ds(c * 16, 16)] * w)
            plsc.parallel_loop(0, K, 1, unroll=4)(_row)

            pltpu.async_copy(rows_v.at[b], acc_sp.at[dst_sb.at[j]],
                             ssem, add=True)
            return c1
        lax.fori_loop(0, SB, _chunk, 0)

        pltpu.make_async_copy(rows_v.at[(SB - 1) % 2],
                              acc_sp.at[dst_sb.at[SB - 1]], ssem).wait()
        return carry
    lax.fori_loop(0, NSB, _sblock, 0)

    pltpu.sync_copy(s_loc, s_hbm.at[wid])
    plsc.subcore_barrier()

    def _wb(t, carry):
        pltpu.sync_copy(
            acc_sp.at[pl.ds(sid * ROWS_PER_TILE + t * ZR, ZR)],
            out_hbm.at[cid, pl.ds(sid * ROWS_PER_TILE + t * ZR, ZR)])
        return carry
    lax.fori_loop(0, ROWS_PER_TILE // ZR, _wb, 0)


_sc_edge = functools.partial(
    pl.kernel,
    out_type=(jax.ShapeDtypeStruct((2, NP, D), jnp.float32),
              jax.ShapeDtypeStruct((NW, AL_R, 128), jnp.float32)),
    mesh=plsc.VectorSubcoreMesh(core_axis_name="c", subcore_axis_name="s"),
    scratch_types=[
        pltpu.VMEM((SB, K), jnp.int32),
        pltpu.VMEM((SB, K), jnp.int32),
        pltpu.VMEM((AL_R, 128), jnp.float32),
        pltpu.VMEM((AL_R, 128), jnp.float32),
        pltpu.VMEM((K // 16, 16), jnp.float32),
        pltpu.VMEM((2, K, D), jnp.float32),
        pltpu.VMEM((AL_R, 128), jnp.float32),
        pltpu.VMEM_SHARED((NP, D), jnp.float32),
        pltpu.SemaphoreType.DMA,
        pltpu.SemaphoreType.DMA,
    ],
    compiler_params=pltpu.CompilerParams(needs_layout_passes=False),
    )(_sc_edge_body)


def kernel(x, edge_index, W1s, W1d, a1s, a1d, b1, W2, a2s, a2d, b2,
           W3, a3s, a3d, b3, lin1_W, lin1_b, lin2_W, lin2_b):
    f32 = jnp.float32
    x_p = jnp.zeros((NP, D), f32).at[:N_NODES].set(x)
    pad = jnp.full((EP - N_EDGES,), PAD_NODE, jnp.int32)
    src = jnp.concatenate([edge_index[0], pad]).reshape(EP // K, K)
    dst = jnp.concatenate([edge_index[1], pad]).reshape(EP // K, K)

    def col(a):
        return a.reshape(D, 1)

    def row(a, w=D):
        return a.reshape(1, w)

    def tr(sv):
        svt = sv.reshape(NW, AL_R * 128).transpose(1, 0)
        return jnp.zeros((NP, NW), jnp.float32).at[:AL_R * 128].set(svt)

    hs, als, ald = _tc_first(x_p, W1s, W1d, col(a1s), col(a1d))
    acc, sv = _sc_edge(hs, src, dst, als, ald)
    hs, als, ald = _tc_mid(acc, tr(sv), row(b1), W2, col(a2s), col(a2d))
    acc, sv = _sc_edge(hs, src, dst, als, ald)
    hs, als, ald = _tc_mid(acc, tr(sv), row(b2), W3, col(a3s), col(a3d))
    acc, sv = _sc_edge(hs, src, dst, als, ald)

    w2p = jnp.zeros((D, D), f32).at[:, :D_OUT].set(lin2_W)
    b2p = jnp.zeros((D,), f32).at[:D_OUT].set(lin2_b)
    out = _tc_last(acc, tr(sv), row(b3), lin1_W, row(lin1_b), w2p, row(b2p))
    return out[:N_NODES, :D_OUT]


# speedup vs baseline: 1.3042x; 1.1101x over previous
import functools

import jax
import jax.numpy as jnp
from jax import lax
from jax.experimental import pallas as pl
from jax.experimental.pallas import tpu as pltpu
from jax.experimental.pallas import tpu_sc as plsc

N_NODES = 10000
N_EDGES = 320000
D = 128
D_OUT = 64

NP = 10240
EP = 327680
PAD_NODE = 10100

NW = 32
EDGES_PER_TILE = EP // NW
K = 64
SB = 4
CH0 = 204
CH1 = 116
AL_R = 80
CHUNKS = EDGES_PER_TILE // K
ROWS_PER_TILE = NP // 16
ZR = 128

_R = 2048
_G = NP // _R
_AR = _R // D


def _tc_first_body(x_ref, ws_ref, wd_ref, as_ref, ad_ref, hs_ref, als_ref, ald_ref):
    x = x_ref[...]
    hs = jnp.dot(x, ws_ref[...], preferred_element_type=jnp.float32)
    hs_ref[...] = hs
    als = jnp.dot(hs, as_ref[...], preferred_element_type=jnp.float32)
    als_ref[...] = als.reshape(_AR, D)
    v = jnp.dot(wd_ref[...], ad_ref[...], preferred_element_type=jnp.float32)
    ald_ref[...] = jnp.dot(x, v, preferred_element_type=jnp.float32).reshape(_AR, D)


def _tc_mid_body(acc_ref, s_ref, b_ref, w_ref, as_ref, ad_ref, hs_ref, als_ref, ald_ref):
    num = acc_ref[0] + acc_ref[1]
    s = jnp.sum(s_ref[...], axis=1, keepdims=True)
    h = jnp.maximum(jnp.where(s > 0.0, num / s, 0.0) + b_ref[...], 0.0)
    hs = jnp.dot(h, w_ref[...], preferred_element_type=jnp.float32)
    hs_ref[...] = hs
    als = jnp.dot(hs, as_ref[...], preferred_element_type=jnp.float32)
    als_ref[...] = als.reshape(_AR, D)
    v = jnp.dot(w_ref[...], ad_ref[...], preferred_element_type=jnp.float32)
    ald_ref[...] = jnp.dot(h, v, preferred_element_type=jnp.float32).reshape(_AR, D)


def _tc_last_body(acc_ref, s_ref, b_ref, w1_ref, b1_ref, w2_ref, b2_ref, out_ref):
    num = acc_ref[0] + acc_ref[1]
    s = jnp.sum(s_ref[...], axis=1, keepdims=True)
    h = jnp.maximum(jnp.where(s > 0.0, num / s, 0.0) + b_ref[...], 0.0)
    h = jnp.maximum(jnp.dot(h, w1_ref[...], preferred_element_type=jnp.float32)
                    + b1_ref[...], 0.0)
    out_ref[...] = jnp.dot(h, w2_ref[...], preferred_element_type=jnp.float32) + b2_ref[...]


def _row_blk(i):
    return (i, 0)


def _acc_blk(i):
    return (0, i, 0)


def _full_blk(i):
    return (0, 0)


_W_SPEC = pl.BlockSpec((D, D), _full_blk)
_A_SPEC = pl.BlockSpec((D, 1), _full_blk)
_B_SPEC = pl.BlockSpec((1, D), _full_blk)
_H_SPEC = pl.BlockSpec((_R, D), _row_blk)
_AL_SPEC = pl.BlockSpec((_AR, D), _row_blk)
_ACC_SPEC = pl.BlockSpec((2, _R, D), _acc_blk)
_S_SPEC = pl.BlockSpec((_R, NW), _row_blk)

_PROJ_OUT = (jax.ShapeDtypeStruct((NP, D), jnp.float32),
             jax.ShapeDtypeStruct((NP // D, D), jnp.float32),
             jax.ShapeDtypeStruct((NP // D, D), jnp.float32))

_tc_first = pl.pallas_call(
    _tc_first_body, grid=(_G,),
    in_specs=[_H_SPEC, _W_SPEC, _W_SPEC, _A_SPEC, _A_SPEC],
    out_specs=[_H_SPEC, _AL_SPEC, _AL_SPEC],
    out_shape=_PROJ_OUT)

_tc_mid = pl.pallas_call(
    _tc_mid_body, grid=(_G,),
    in_specs=[_ACC_SPEC, _S_SPEC, _B_SPEC, _W_SPEC, _A_SPEC, _A_SPEC],
    out_specs=[_H_SPEC, _AL_SPEC, _AL_SPEC],
    out_shape=_PROJ_OUT)

_tc_last = pl.pallas_call(
    _tc_last_body, grid=(_G,),
    in_specs=[_ACC_SPEC, _S_SPEC, _B_SPEC, _W_SPEC, _B_SPEC, _W_SPEC, _B_SPEC],
    out_specs=_H_SPEC,
    out_shape=jax.ShapeDtypeStruct((NP, D), jnp.float32))


def _sc_edge_body(hs_hbm, src_hbm, dst_hbm, as_hbm, ad_hbm, out_hbm, s_hbm,
                  src_sb, dst_sb, as_v, ad_v, ee_v, rows_v, s_loc, acc_sp,
                  gsem, ssem):
    cid = lax.axis_index("c")
    sid = lax.axis_index("s")
    tile_base = sid * (CH0 + CH1) + cid * CH0
    nsb = jnp.where(cid == 0, CH0 // SB, CH1 // SB)

    pltpu.sync_copy(as_hbm.at[pl.ds(0, AL_R)], as_v)
    pltpu.sync_copy(ad_hbm.at[pl.ds(0, AL_R)], ad_v)

    zero16 = jnp.zeros((16,), jnp.float32)

    def _z(r, carry):
        for c in range(D // 16):
            rows_v[0, r, pl.ds(c * 16, 16)] = zero16
        return carry
    lax.fori_loop(0, K, _z, 0)

    def _zs(r, carry):
        for c in range(128 // 16):
            s_loc[r, pl.ds(c * 16, 16)] = zero16
        return carry
    lax.fori_loop(0, AL_R, _zs, 0)

    def _za(t, carry):
        pltpu.async_copy(rows_v.at[0],
                         acc_sp.at[pl.ds(sid * ROWS_PER_TILE + t * K, K)], gsem)
        return carry
    lax.fori_loop(0, ROWS_PER_TILE // K, _za, 0)

    def _zw(t, carry):
        pltpu.make_async_copy(
            rows_v.at[0],
            acc_sp.at[pl.ds(sid * ROWS_PER_TILE + t * K, K)], gsem).wait()
        return carry
    lax.fori_loop(0, ROWS_PER_TILE // K, _zw, 0)
    plsc.subcore_barrier()

    def _sblock(sb, carry):
        base = tile_base + sb * SB
        pltpu.sync_copy(src_hbm.at[pl.ds(base, SB)], src_sb)
        pltpu.sync_copy(dst_hbm.at[pl.ds(base, SB)], dst_sb)

        pltpu.async_copy(hs_hbm.at[src_sb.at[0]], rows_v.at[0], gsem)

        def _chunk(j, c1):
            b = j % 2
            def _ee(t, c2):
                sv = src_sb[j, pl.ds(t * 16, 16)]
                dv = dst_sb[j, pl.ds(t * 16, 16)]
                e = (plsc.load_gather(as_v, [sv >> 7, sv & 127])
                     + plsc.load_gather(ad_v, [dv >> 7, dv & 127]))
                e = jnp.where(e > 0.0, e, 0.2 * e)
                ee = jnp.exp(e)
                ee_v[t] = ee
                plsc.addupdate_scatter(s_loc, [dv >> 7, dv & 127], ee)
                return c2
            lax.fori_loop(0, K // 16, _ee, 0)

            pltpu.make_async_copy(hs_hbm.at[src_sb.at[j]],
                                  rows_v.at[b], gsem).wait()

            @pl.when(j >= 1)
            def _():
                pltpu.make_async_copy(rows_v.at[1 - b],
                                      acc_sp.at[dst_sb.at[j - 1]],
                                      ssem).wait()

            @pl.when(j < SB - 1)
            def _():
                pltpu.async_copy(hs_hbm.at[src_sb.at[j + 1]],
                                 rows_v.at[1 - b], gsem)

            def _row(i):
                w = plsc.load_gather(
                    ee_v, [jnp.broadcast_to(i >> 4, (16,)),
                           jnp.broadcast_to(i & 15, (16,))])
                for c in range(D // 16):
                    rows_v[b, i, pl.ds(c * 16, 16)] = (
                        rows_v[b, i, pl.ds(c * 16, 16)] * w)
            plsc.parallel_loop(0, K, 1, unroll=4)(_row)

            pltpu.async_copy(rows_v.at[b], acc_sp.at[dst_sb.at[j]],
                             ssem, add=True)
            return c1
        lax.fori_loop(0, SB, _chunk, 0)

        pltpu.make_async_copy(rows_v.at[(SB - 1) % 2],
                              acc_sp.at[dst_sb.at[SB - 1]], ssem).wait()
        return carry
    lax.fori_loop(0, nsb, _sblock, 0)

    pltpu.sync_copy(s_loc, s_hbm.at[sid * 2 + cid])
    plsc.subcore_barrier()

    def _wb(t, carry):
        pltpu.sync_copy(
            acc_sp.at[pl.ds(sid * ROWS_PER_TILE + t * ZR, ZR)],
            out_hbm.at[cid, pl.ds(sid * ROWS_PER_TILE + t * ZR, ZR)])
        return carry
    lax.fori_loop(0, ROWS_PER_TILE // ZR, _wb, 0)


_sc_edge = functools.partial(
    pl.kernel,
    out_type=(jax.ShapeDtypeStruct((2, NP, D), jnp.float32),
              jax.ShapeDtypeStruct((NW, AL_R, 128), jnp.float32)),
    mesh=plsc.VectorSubcoreMesh(core_axis_name="c", subcore_axis_name="s"),
    scratch_types=[
        pltpu.VMEM((SB, K), jnp.int32),
        pltpu.VMEM((SB, K), jnp.int32),
        pltpu.VMEM((AL_R, 128), jnp.float32),
        pltpu.VMEM((AL_R, 128), jnp.float32),
        pltpu.VMEM((K // 16, 16), jnp.float32),
        pltpu.VMEM((2, K, D), jnp.float32),
        pltpu.VMEM((AL_R, 128), jnp.float32),
        pltpu.VMEM_SHARED((NP, D), jnp.float32),
        pltpu.SemaphoreType.DMA,
        pltpu.SemaphoreType.DMA,
    ],
    compiler_params=pltpu.CompilerParams(needs_layout_passes=False),
    )(_sc_edge_body)


def kernel(x, edge_index, W1s, W1d, a1s, a1d, b1, W2, a2s, a2d, b2,
           W3, a3s, a3d, b3, lin1_W, lin1_b, lin2_W, lin2_b):
    f32 = jnp.float32
    x_p = jnp.zeros((NP, D), f32).at[:N_NODES].set(x)
    pad = jnp.full((EP - N_EDGES,), PAD_NODE, jnp.int32)
    src = jnp.concatenate([edge_index[0], pad]).reshape(EP // K, K)
    dst = jnp.concatenate([edge_index[1], pad]).reshape(EP // K, K)

    def col(a):
        return a.reshape(D, 1)

    def row(a, w=D):
        return a.reshape(1, w)

    def tr(sv):
        svt = sv.reshape(NW, AL_R * 128).transpose(1, 0)
        return jnp.zeros((NP, NW), jnp.float32).at[:AL_R * 128].set(svt)

    hs, als, ald = _tc_first(x_p, W1s, W1d, col(a1s), col(a1d))
    acc, sv = _sc_edge(hs, src, dst, als, ald)
    hs, als, ald = _tc_mid(acc, tr(sv), row(b1), W2, col(a2s), col(a2d))
    acc, sv = _sc_edge(hs, src, dst, als, ald)
    hs, als, ald = _tc_mid(acc, tr(sv), row(b2), W3, col(a3s), col(a3d))
    acc, sv = _sc_edge(hs, src, dst, als, ald)

    w2p = jnp.zeros((D, D), f32).at[:, :D_OUT].set(lin2_W)
    b2p = jnp.zeros((D,), f32).at[:D_OUT].set(lin2_b)
    out = _tc_last(acc, tr(sv), row(b3), lin1_W, row(lin1_b), w2p, row(b2p))
    return out[:N_NODES, :D_OUT]


# speedup vs baseline: 1.3047x; 1.0004x over previous
import functools

import jax
import jax.numpy as jnp
from jax import lax
from jax.experimental import pallas as pl
from jax.experimental.pallas import tpu as pltpu
from jax.experimental.pallas import tpu_sc as plsc

N_NODES = 10000
N_EDGES = 320000
D = 128
D_OUT = 64

NP = 10240
EP = 327680
PAD_NODE = 10100

NW = 32
EDGES_PER_TILE = EP // NW
K = 64
SB = 4
CH0 = 204
CH1 = 116
AL_R = 80
CHUNKS = EDGES_PER_TILE // K
ROWS_PER_TILE = NP // 16
ZR = 128

_R = 2048
_G = NP // _R
_AR = _R // D


def _tc_first_body(x_ref, ws_ref, wd_ref, as_ref, ad_ref, hs_ref, als_ref, ald_ref):
    x = x_ref[...]
    hs = jnp.dot(x, ws_ref[...], preferred_element_type=jnp.float32)
    hs_ref[...] = hs
    als = jnp.dot(hs, as_ref[...], preferred_element_type=jnp.float32)
    als_ref[...] = als.reshape(_AR, D)
    v = jnp.dot(wd_ref[...], ad_ref[...], preferred_element_type=jnp.float32)
    ald_ref[...] = jnp.dot(x, v, preferred_element_type=jnp.float32).reshape(_AR, D)


def _tc_mid_body(acc_ref, s_ref, b_ref, w_ref, as_ref, ad_ref, hs_ref, als_ref, ald_ref):
    num = acc_ref[0] + acc_ref[1]
    s = jnp.sum(s_ref[...], axis=1, keepdims=True)
    h = jnp.maximum(jnp.where(s > 0.0, num / s, 0.0) + b_ref[...], 0.0)
    hs = jnp.dot(h, w_ref[...], preferred_element_type=jnp.float32)
    hs_ref[...] = hs
    als = jnp.dot(hs, as_ref[...], preferred_element_type=jnp.float32)
    als_ref[...] = als.reshape(_AR, D)
    v = jnp.dot(w_ref[...], ad_ref[...], preferred_element_type=jnp.float32)
    ald_ref[...] = jnp.dot(h, v, preferred_element_type=jnp.float32).reshape(_AR, D)


def _tc_last_body(acc_ref, s_ref, b_ref, w1_ref, b1_ref, w2_ref, b2_ref, out_ref):
    num = acc_ref[0] + acc_ref[1]
    s = jnp.sum(s_ref[...], axis=1, keepdims=True)
    h = jnp.maximum(jnp.where(s > 0.0, num / s, 0.0) + b_ref[...], 0.0)
    h = jnp.maximum(jnp.dot(h, w1_ref[...], preferred_element_type=jnp.float32)
                    + b1_ref[...], 0.0)
    out_ref[...] = jnp.dot(h, w2_ref[...], preferred_element_type=jnp.float32) + b2_ref[...]


def _row_blk(i):
    return (i, 0)


def _acc_blk(i):
    return (0, i, 0)


def _full_blk(i):
    return (0, 0)


_W_SPEC = pl.BlockSpec((D, D), _full_blk)
_A_SPEC = pl.BlockSpec((D, 1), _full_blk)
_B_SPEC = pl.BlockSpec((1, D), _full_blk)
_H_SPEC = pl.BlockSpec((_R, D), _row_blk)
_AL_SPEC = pl.BlockSpec((_AR, D), _row_blk)
_ACC_SPEC = pl.BlockSpec((2, _R, D), _acc_blk)
_S_SPEC = pl.BlockSpec((_R, NW), _row_blk)

_PROJ_OUT = (jax.ShapeDtypeStruct((NP, D), jnp.float32),
             jax.ShapeDtypeStruct((NP // D, D), jnp.float32),
             jax.ShapeDtypeStruct((NP // D, D), jnp.float32))

_tc_first = pl.pallas_call(
    _tc_first_body, grid=(_G,),
    in_specs=[_H_SPEC, _W_SPEC, _W_SPEC, _A_SPEC, _A_SPEC],
    out_specs=[_H_SPEC, _AL_SPEC, _AL_SPEC],
    out_shape=_PROJ_OUT)

_tc_mid = pl.pallas_call(
    _tc_mid_body, grid=(_G,),
    in_specs=[_ACC_SPEC, _S_SPEC, _B_SPEC, _W_SPEC, _A_SPEC, _A_SPEC],
    out_specs=[_H_SPEC, _AL_SPEC, _AL_SPEC],
    out_shape=_PROJ_OUT)

_tc_last = pl.pallas_call(
    _tc_last_body, grid=(_G,),
    in_specs=[_ACC_SPEC, _S_SPEC, _B_SPEC, _W_SPEC, _B_SPEC, _W_SPEC, _B_SPEC],
    out_specs=_H_SPEC,
    out_shape=jax.ShapeDtypeStruct((NP, D), jnp.float32))


def _sc_edge_body(hs_hbm, src_hbm, dst_hbm, as_hbm, ad_hbm, out_hbm, s_hbm,
                  src_sb, dst_sb, as_v, ad_v, ee_v, rows_v, s_loc, acc_sp,
                  gsem, ssem):
    cid = lax.axis_index("c")
    sid = lax.axis_index("s")
    tile_base = sid * (CH0 + CH1) + cid * CH0
    nsb = jnp.where(cid == 0, CH0 // SB, CH1 // SB)

    with jax.named_scope("stage_alpha"):
        pltpu.sync_copy(as_hbm.at[pl.ds(0, AL_R)], as_v)
        pltpu.sync_copy(ad_hbm.at[pl.ds(0, AL_R)], ad_v)

    zero16 = jnp.zeros((16,), jnp.float32)

    def _z(r, carry):
        for c in range(D // 16):
            rows_v[0, r, pl.ds(c * 16, 16)] = zero16
        return carry
    lax.fori_loop(0, K, _z, 0)

    def _zs(r, carry):
        for c in range(128 // 16):
            s_loc[r, pl.ds(c * 16, 16)] = zero16
        return carry
    lax.fori_loop(0, AL_R, _zs, 0)

    def _za(t, carry):
        pltpu.async_copy(rows_v.at[0],
                         acc_sp.at[pl.ds(sid * ROWS_PER_TILE + t * K, K)], gsem)
        return carry
    lax.fori_loop(0, ROWS_PER_TILE // K, _za, 0)

    def _zw(t, carry):
        pltpu.make_async_copy(
            rows_v.at[0],
            acc_sp.at[pl.ds(sid * ROWS_PER_TILE + t * K, K)], gsem).wait()
        return carry
    with jax.named_scope("zero_drain"):
        lax.fori_loop(0, ROWS_PER_TILE // K, _zw, 0)
    plsc.subcore_barrier()

    def _sblock(sb, carry):
        base = tile_base + sb * SB
        pltpu.sync_copy(src_hbm.at[pl.ds(base, SB)], src_sb)
        pltpu.sync_copy(dst_hbm.at[pl.ds(base, SB)], dst_sb)

        pltpu.async_copy(hs_hbm.at[src_sb.at[0]], rows_v.at[0], gsem)

        def _chunk(j, c1):
            b = j % 2
            def _ee(t, c2):
                sv = src_sb[j, pl.ds(t * 16, 16)]
                dv = dst_sb[j, pl.ds(t * 16, 16)]
                e = (plsc.load_gather(as_v, [sv >> 7, sv & 127])
                     + plsc.load_gather(ad_v, [dv >> 7, dv & 127]))
                e = jnp.where(e > 0.0, e, 0.2 * e)
                ee = jnp.exp(e)
                ee_v[t] = ee
                plsc.addupdate_scatter(s_loc, [dv >> 7, dv & 127], ee)
                return c2
            lax.fori_loop(0, K // 16, _ee, 0)

            pltpu.make_async_copy(hs_hbm.at[src_sb.at[j]],
                                  rows_v.at[b], gsem).wait()

            @pl.when(j >= 1)
            def _():
                pltpu.make_async_copy(rows_v.at[1 - b],
                                      acc_sp.at[dst_sb.at[j - 1]],
                                      ssem).wait()

            @pl.when(j < SB - 1)
            def _():
                pltpu.async_copy(hs_hbm.at[src_sb.at[j + 1]],
                                 rows_v.at[1 - b], gsem)

            def _row(i):
                w = plsc.load_gather(
                    ee_v, [jnp.broadcast_to(i >> 4, (16,)),
                           jnp.broadcast_to(i & 15, (16,))])
                for c in range(D // 16):
                    rows_v[b, i, pl.ds(c * 16, 16)] = (
                        rows_v[b, i, pl.ds(c * 16, 16)] * w)
            plsc.parallel_loop(0, K, 1, unroll=4)(_row)

            pltpu.async_copy(rows_v.at[b], acc_sp.at[dst_sb.at[j]],
                             ssem, add=True)
            return c1
        lax.fori_loop(0, SB, _chunk, 0)

        pltpu.make_async_copy(rows_v.at[(SB - 1) % 2],
                              acc_sp.at[dst_sb.at[SB - 1]], ssem).wait()
        return carry
    with jax.named_scope("edge_main"):
        lax.fori_loop(0, nsb, _sblock, 0)

    pltpu.sync_copy(s_loc, s_hbm.at[sid * 2 + cid])
    plsc.subcore_barrier()

    with jax.named_scope("writeback"):
        def _wb(t, carry):
            pltpu.sync_copy(
                acc_sp.at[pl.ds(sid * ROWS_PER_TILE + t * ZR, ZR)],
                out_hbm.at[cid, pl.ds(sid * ROWS_PER_TILE + t * ZR, ZR)])
            return carry
        lax.fori_loop(0, ROWS_PER_TILE // ZR, _wb, 0)


_sc_edge = functools.partial(
    pl.kernel,
    out_type=(jax.ShapeDtypeStruct((2, NP, D), jnp.float32),
              jax.ShapeDtypeStruct((NW, AL_R, 128), jnp.float32)),
    mesh=plsc.VectorSubcoreMesh(core_axis_name="c", subcore_axis_name="s"),
    scratch_types=[
        pltpu.VMEM((SB, K), jnp.int32),
        pltpu.VMEM((SB, K), jnp.int32),
        pltpu.VMEM((AL_R, 128), jnp.float32),
        pltpu.VMEM((AL_R, 128), jnp.float32),
        pltpu.VMEM((K // 16, 16), jnp.float32),
        pltpu.VMEM((2, K, D), jnp.float32),
        pltpu.VMEM((AL_R, 128), jnp.float32),
        pltpu.VMEM_SHARED((NP, D), jnp.float32),
        pltpu.SemaphoreType.DMA,
        pltpu.SemaphoreType.DMA,
    ],
    compiler_params=pltpu.CompilerParams(needs_layout_passes=False),
    )(_sc_edge_body)


def kernel(x, edge_index, W1s, W1d, a1s, a1d, b1, W2, a2s, a2d, b2,
           W3, a3s, a3d, b3, lin1_W, lin1_b, lin2_W, lin2_b):
    f32 = jnp.float32
    x_p = jnp.zeros((NP, D), f32).at[:N_NODES].set(x)
    pad = jnp.full((EP - N_EDGES,), PAD_NODE, jnp.int32)
    src = jnp.concatenate([edge_index[0], pad]).reshape(EP // K, K)
    dst = jnp.concatenate([edge_index[1], pad]).reshape(EP // K, K)

    def col(a):
        return a.reshape(D, 1)

    def row(a, w=D):
        return a.reshape(1, w)

    def tr(sv):
        svt = sv.reshape(NW, AL_R * 128).transpose(1, 0)
        return jnp.zeros((NP, NW), jnp.float32).at[:AL_R * 128].set(svt)

    hs, als, ald = _tc_first(x_p, W1s, W1d, col(a1s), col(a1d))
    acc, sv = _sc_edge(hs, src, dst, als, ald)
    hs, als, ald = _tc_mid(acc, tr(sv), row(b1), W2, col(a2s), col(a2d))
    acc, sv = _sc_edge(hs, src, dst, als, ald)
    hs, als, ald = _tc_mid(acc, tr(sv), row(b2), W3, col(a3s), col(a3d))
    acc, sv = _sc_edge(hs, src, dst, als, ald)

    w2p = jnp.zeros((D, D), f32).at[:, :D_OUT].set(lin2_W)
    b2p = jnp.zeros((D,), f32).at[:D_OUT].set(lin2_b)
    out = _tc_last(acc, tr(sv), row(b3), lin1_W, row(lin1_b), w2p, row(b2p))
    return out[:N_NODES, :D_OUT]


# speedup vs baseline: 1.3893x; 1.0648x over previous
import functools

import jax
import jax.numpy as jnp
from jax import lax
from jax.experimental import pallas as pl
from jax.experimental.pallas import tpu as pltpu
from jax.experimental.pallas import tpu_sc as plsc

N_NODES = 10000
N_EDGES = 320000
D = 128
D_OUT = 64

NP = 10240
EP = 327680
PAD_NODE = 10100

NW = 32
EDGES_PER_TILE = EP // NW
K = 64
SB = 4
CH0 = 240
CH1 = 80
AL_R = 80
CHUNKS = EDGES_PER_TILE // K
ROWS_PER_TILE = NP // 16
ZR = 128

_R = 2048
_G = NP // _R
_AR = _R // D


def _tc_first_body(x_ref, ws_ref, wd_ref, as_ref, ad_ref, hs_ref, als_ref, ald_ref):
    x = x_ref[...]
    hs = jnp.dot(x, ws_ref[...], preferred_element_type=jnp.float32)
    hs_ref[...] = hs
    als = jnp.dot(hs, as_ref[...], preferred_element_type=jnp.float32)
    als_ref[...] = als.reshape(_AR, D)
    v = jnp.dot(wd_ref[...], ad_ref[...], preferred_element_type=jnp.float32)
    ald_ref[...] = jnp.dot(x, v, preferred_element_type=jnp.float32).reshape(_AR, D)


def _tc_mid_body(acc_ref, s_ref, b_ref, w_ref, as_ref, ad_ref, hs_ref, als_ref, ald_ref):
    num = acc_ref[0] + acc_ref[1]
    s = jnp.sum(s_ref[...], axis=1, keepdims=True)
    h = jnp.maximum(jnp.where(s > 0.0, num / s, 0.0) + b_ref[...], 0.0)
    hs = jnp.dot(h, w_ref[...], preferred_element_type=jnp.float32)
    hs_ref[...] = hs
    als = jnp.dot(hs, as_ref[...], preferred_element_type=jnp.float32)
    als_ref[...] = als.reshape(_AR, D)
    v = jnp.dot(w_ref[...], ad_ref[...], preferred_element_type=jnp.float32)
    ald_ref[...] = jnp.dot(h, v, preferred_element_type=jnp.float32).reshape(_AR, D)


def _tc_last_body(acc_ref, s_ref, b_ref, w1_ref, b1_ref, w2_ref, b2_ref, out_ref):
    num = acc_ref[0] + acc_ref[1]
    s = jnp.sum(s_ref[...], axis=1, keepdims=True)
    h = jnp.maximum(jnp.where(s > 0.0, num / s, 0.0) + b_ref[...], 0.0)
    h = jnp.maximum(jnp.dot(h, w1_ref[...], preferred_element_type=jnp.float32)
                    + b1_ref[...], 0.0)
    out_ref[...] = jnp.dot(h, w2_ref[...], preferred_element_type=jnp.float32) + b2_ref[...]


def _row_blk(i):
    return (i, 0)


def _acc_blk(i):
    return (0, i, 0)


def _full_blk(i):
    return (0, 0)


_W_SPEC = pl.BlockSpec((D, D), _full_blk)
_A_SPEC = pl.BlockSpec((D, 1), _full_blk)
_B_SPEC = pl.BlockSpec((1, D), _full_blk)
_H_SPEC = pl.BlockSpec((_R, D), _row_blk)
_AL_SPEC = pl.BlockSpec((_AR, D), _row_blk)
_ACC_SPEC = pl.BlockSpec((2, _R, D), _acc_blk)
_S_SPEC = pl.BlockSpec((_R, NW), _row_blk)

_PROJ_OUT = (jax.ShapeDtypeStruct((NP, D), jnp.float32),
             jax.ShapeDtypeStruct((NP // D, D), jnp.float32),
             jax.ShapeDtypeStruct((NP // D, D), jnp.float32))

_tc_first = pl.pallas_call(
    _tc_first_body, grid=(_G,),
    in_specs=[_H_SPEC, _W_SPEC, _W_SPEC, _A_SPEC, _A_SPEC],
    out_specs=[_H_SPEC, _AL_SPEC, _AL_SPEC],
    out_shape=_PROJ_OUT)

_tc_mid = pl.pallas_call(
    _tc_mid_body, grid=(_G,),
    in_specs=[_ACC_SPEC, _S_SPEC, _B_SPEC, _W_SPEC, _A_SPEC, _A_SPEC],
    out_specs=[_H_SPEC, _AL_SPEC, _AL_SPEC],
    out_shape=_PROJ_OUT)

_tc_last = pl.pallas_call(
    _tc_last_body, grid=(_G,),
    in_specs=[_ACC_SPEC, _S_SPEC, _B_SPEC, _W_SPEC, _B_SPEC, _W_SPEC, _B_SPEC],
    out_specs=_H_SPEC,
    out_shape=jax.ShapeDtypeStruct((NP, D), jnp.float32))


def _sc_edge_body(hs_hbm, src_hbm, dst_hbm, as_hbm, ad_hbm, out_hbm, s_hbm,
                  src_sb, dst_sb, as_v, ad_v, ee_v, rows_v, s_loc, acc_sp,
                  gsem, ssem):
    cid = lax.axis_index("c")
    sid = lax.axis_index("s")
    tile_base = sid * (CH0 + CH1) + cid * CH0
    nsb = jnp.where(cid == 0, CH0 // SB, CH1 // SB)

    pltpu.sync_copy(as_hbm.at[pl.ds(0, AL_R)], as_v)
    pltpu.sync_copy(ad_hbm.at[pl.ds(0, AL_R)], ad_v)

    zero16 = jnp.zeros((16,), jnp.float32)

    def _z(r, carry):
        for c in range(D // 16):
            rows_v[0, r, pl.ds(c * 16, 16)] = zero16
        return carry
    lax.fori_loop(0, K, _z, 0)

    def _zs(r, carry):
        for c in range(128 // 16):
            s_loc[r, pl.ds(c * 16, 16)] = zero16
        return carry
    lax.fori_loop(0, AL_R, _zs, 0)

    def _za(t, carry):
        pltpu.async_copy(rows_v.at[0],
                         acc_sp.at[pl.ds(sid * ROWS_PER_TILE + t * K, K)], gsem)
        return carry
    lax.fori_loop(0, ROWS_PER_TILE // K, _za, 0)

    def _zw(t, carry):
        pltpu.make_async_copy(
            rows_v.at[0],
            acc_sp.at[pl.ds(sid * ROWS_PER_TILE + t * K, K)], gsem).wait()
        return carry
    lax.fori_loop(0, ROWS_PER_TILE // K, _zw, 0)
    plsc.subcore_barrier()

    def _sblock(sb, carry):
        base = tile_base + sb * SB
        pltpu.sync_copy(src_hbm.at[pl.ds(base, SB)], src_sb)
        pltpu.sync_copy(dst_hbm.at[pl.ds(base, SB)], dst_sb)

        pltpu.async_copy(hs_hbm.at[src_sb.at[0]], rows_v.at[0], gsem)

        def _chunk(j, c1):
            b = j % 2
            @pl.when(j >= 1)
            def _():
                pltpu.make_async_copy(rows_v.at[1 - b],
                                      acc_sp.at[dst_sb.at[j - 1]],
                                      ssem).wait()

            @pl.when(j < SB - 1)
            def _():
                pltpu.async_copy(hs_hbm.at[src_sb.at[j + 1]],
                                 rows_v.at[1 - b], gsem)

            def _ee(t, c2):
                sv = src_sb[j, pl.ds(t * 16, 16)]
                dv = dst_sb[j, pl.ds(t * 16, 16)]
                e = (plsc.load_gather(as_v, [sv >> 7, sv & 127])
                     + plsc.load_gather(ad_v, [dv >> 7, dv & 127]))
                e = jnp.where(e > 0.0, e, 0.2 * e)
                ee = jnp.exp(e)
                ee_v[t] = ee
                plsc.addupdate_scatter(s_loc, [dv >> 7, dv & 127], ee)
                return c2
            lax.fori_loop(0, K // 16, _ee, 0)

            pltpu.make_async_copy(hs_hbm.at[src_sb.at[j]],
                                  rows_v.at[b], gsem).wait()

            def _row(i):
                w = plsc.load_gather(
                    ee_v, [jnp.broadcast_to(i >> 4, (16,)),
                           jnp.broadcast_to(i & 15, (16,))])
                for c in range(D // 16):
                    rows_v[b, i, pl.ds(c * 16, 16)] = (
                        rows_v[b, i, pl.ds(c * 16, 16)] * w)
            plsc.parallel_loop(0, K, 1, unroll=4)(_row)

            pltpu.async_copy(rows_v.at[b], acc_sp.at[dst_sb.at[j]],
                             ssem, add=True)
            return c1
        lax.fori_loop(0, SB, _chunk, 0)

        pltpu.make_async_copy(rows_v.at[(SB - 1) % 2],
                              acc_sp.at[dst_sb.at[SB - 1]], ssem).wait()
        return carry
    lax.fori_loop(0, nsb, _sblock, 0)

    pltpu.sync_copy(s_loc, s_hbm.at[sid * 2 + cid])
    plsc.subcore_barrier()

    def _wb(t, carry):
        pltpu.sync_copy(
            acc_sp.at[pl.ds(sid * ROWS_PER_TILE + t * ZR, ZR)],
            out_hbm.at[cid, pl.ds(sid * ROWS_PER_TILE + t * ZR, ZR)])
        return carry
    lax.fori_loop(0, ROWS_PER_TILE // ZR, _wb, 0)


_sc_edge = functools.partial(
    pl.kernel,
    out_type=(jax.ShapeDtypeStruct((2, NP, D), jnp.float32),
              jax.ShapeDtypeStruct((NW, AL_R, 128), jnp.float32)),
    mesh=plsc.VectorSubcoreMesh(core_axis_name="c", subcore_axis_name="s"),
    scratch_types=[
        pltpu.VMEM((SB, K), jnp.int32),
        pltpu.VMEM((SB, K), jnp.int32),
        pltpu.VMEM((AL_R, 128), jnp.float32),
        pltpu.VMEM((AL_R, 128), jnp.float32),
        pltpu.VMEM((K // 16, 16), jnp.float32),
        pltpu.VMEM((2, K, D), jnp.float32),
        pltpu.VMEM((AL_R, 128), jnp.float32),
        pltpu.VMEM_SHARED((NP, D), jnp.float32),
        pltpu.SemaphoreType.DMA,
        pltpu.SemaphoreType.DMA,
    ],
    compiler_params=pltpu.CompilerParams(needs_layout_passes=False),
    )(_sc_edge_body)


def kernel(x, edge_index, W1s, W1d, a1s, a1d, b1, W2, a2s, a2d, b2,
           W3, a3s, a3d, b3, lin1_W, lin1_b, lin2_W, lin2_b):
    f32 = jnp.float32
    x_p = jnp.zeros((NP, D), f32).at[:N_NODES].set(x)
    pad = jnp.full((EP - N_EDGES,), PAD_NODE, jnp.int32)
    src = jnp.concatenate([edge_index[0], pad]).reshape(EP // K, K)
    dst = jnp.concatenate([edge_index[1], pad]).reshape(EP // K, K)

    def col(a):
        return a.reshape(D, 1)

    def row(a, w=D):
        return a.reshape(1, w)

    def tr(sv):
        svt = sv.reshape(NW, AL_R * 128).transpose(1, 0)
        return jnp.zeros((NP, NW), jnp.float32).at[:AL_R * 128].set(svt)

    hs, als, ald = _tc_first(x_p, W1s, W1d, col(a1s), col(a1d))
    acc, sv = _sc_edge(hs, src, dst, als, ald)
    hs, als, ald = _tc_mid(acc, tr(sv), row(b1), W2, col(a2s), col(a2d))
    acc, sv = _sc_edge(hs, src, dst, als, ald)
    hs, als, ald = _tc_mid(acc, tr(sv), row(b2), W3, col(a3s), col(a3d))
    acc, sv = _sc_edge(hs, src, dst, als, ald)

    w2p = jnp.zeros((D, D), f32).at[:, :D_OUT].set(lin2_W)
    b2p = jnp.zeros((D,), f32).at[:D_OUT].set(lin2_b)
    out = _tc_last(acc, tr(sv), row(b3), lin1_W, row(lin1_b), w2p, row(b2p))
    return out[:N_NODES, :D_OUT]


# speedup vs baseline: 1.4009x; 1.0084x over previous
import functools

import jax
import jax.numpy as jnp
from jax import lax
from jax.experimental import pallas as pl
from jax.experimental.pallas import tpu as pltpu
from jax.experimental.pallas import tpu_sc as plsc

N_NODES = 10000
N_EDGES = 320000
D = 128
D_OUT = 64

NP = 10240
EP = 327680
PAD_NODE = 10100

NW = 32
EDGES_PER_TILE = EP // NW
K = 64
SB = 4
CH0 = 208
CH1 = 112
AL_R = 80
CHUNKS = EDGES_PER_TILE // K
ROWS_PER_TILE = NP // 16
ZR = 128

_R = 2048
_G = NP // _R
_AR = _R // D


def _tc_first_body(x_ref, ws_ref, wd_ref, as_ref, ad_ref, hs_ref, als_ref, ald_ref):
    x = x_ref[...]
    hs = jnp.dot(x, ws_ref[...], preferred_element_type=jnp.float32)
    hs_ref[...] = hs
    als = jnp.dot(hs, as_ref[...], preferred_element_type=jnp.float32)
    als_ref[...] = als.reshape(_AR, D)
    v = jnp.dot(wd_ref[...], ad_ref[...], preferred_element_type=jnp.float32)
    ald_ref[...] = jnp.dot(x, v, preferred_element_type=jnp.float32).reshape(_AR, D)


def _tc_mid_body(acc_ref, s_ref, b_ref, w_ref, as_ref, ad_ref, hs_ref, als_ref, ald_ref):
    num = acc_ref[0] + acc_ref[1]
    s = jnp.sum(s_ref[...], axis=1, keepdims=True)
    h = jnp.maximum(jnp.where(s > 0.0, num / s, 0.0) + b_ref[...], 0.0)
    hs = jnp.dot(h, w_ref[...], preferred_element_type=jnp.float32)
    hs_ref[...] = hs
    als = jnp.dot(hs, as_ref[...], preferred_element_type=jnp.float32)
    als_ref[...] = als.reshape(_AR, D)
    v = jnp.dot(w_ref[...], ad_ref[...], preferred_element_type=jnp.float32)
    ald_ref[...] = jnp.dot(h, v, preferred_element_type=jnp.float32).reshape(_AR, D)


def _tc_last_body(acc_ref, s_ref, b_ref, w1_ref, b1_ref, w2_ref, b2_ref, out_ref):
    num = acc_ref[0] + acc_ref[1]
    s = jnp.sum(s_ref[...], axis=1, keepdims=True)
    h = jnp.maximum(jnp.where(s > 0.0, num / s, 0.0) + b_ref[...], 0.0)
    h = jnp.maximum(jnp.dot(h, w1_ref[...], preferred_element_type=jnp.float32)
                    + b1_ref[...], 0.0)
    out_ref[...] = jnp.dot(h, w2_ref[...], preferred_element_type=jnp.float32) + b2_ref[...]


def _row_blk(i):
    return (i, 0)


def _acc_blk(i):
    return (0, i, 0)


def _full_blk(i):
    return (0, 0)


_W_SPEC = pl.BlockSpec((D, D), _full_blk)
_A_SPEC = pl.BlockSpec((D, 1), _full_blk)
_B_SPEC = pl.BlockSpec((1, D), _full_blk)
_H_SPEC = pl.BlockSpec((_R, D), _row_blk)
_AL_SPEC = pl.BlockSpec((_AR, D), _row_blk)
_ACC_SPEC = pl.BlockSpec((2, _R, D), _acc_blk)
_S_SPEC = pl.BlockSpec((_R, NW), _row_blk)

_PROJ_OUT = (jax.ShapeDtypeStruct((NP, D), jnp.float32),
             jax.ShapeDtypeStruct((NP // D, D), jnp.float32),
             jax.ShapeDtypeStruct((NP // D, D), jnp.float32))

_tc_first = pl.pallas_call(
    _tc_first_body, grid=(_G,),
    in_specs=[_H_SPEC, _W_SPEC, _W_SPEC, _A_SPEC, _A_SPEC],
    out_specs=[_H_SPEC, _AL_SPEC, _AL_SPEC],
    out_shape=_PROJ_OUT)

_tc_mid = pl.pallas_call(
    _tc_mid_body, grid=(_G,),
    in_specs=[_ACC_SPEC, _S_SPEC, _B_SPEC, _W_SPEC, _A_SPEC, _A_SPEC],
    out_specs=[_H_SPEC, _AL_SPEC, _AL_SPEC],
    out_shape=_PROJ_OUT)

_tc_last = pl.pallas_call(
    _tc_last_body, grid=(_G,),
    in_specs=[_ACC_SPEC, _S_SPEC, _B_SPEC, _W_SPEC, _B_SPEC, _W_SPEC, _B_SPEC],
    out_specs=_H_SPEC,
    out_shape=jax.ShapeDtypeStruct((NP, D), jnp.float32))


def _sc_edge_body(hs_hbm, src_hbm, dst_hbm, as_hbm, ad_hbm, out_hbm, s_hbm,
                  src_sb, dst_sb, as_v, ad_v, ee_v, rows_v, s_loc, acc_sp,
                  gsem, ssem):
    cid = lax.axis_index("c")
    sid = lax.axis_index("s")
    tile_base = sid * (CH0 + CH1) + cid * CH0
    nsb = jnp.where(cid == 0, CH0 // SB, CH1 // SB)

    pltpu.sync_copy(as_hbm.at[pl.ds(0, AL_R)], as_v)
    pltpu.sync_copy(ad_hbm.at[pl.ds(0, AL_R)], ad_v)

    zero16 = jnp.zeros((16,), jnp.float32)

    def _z(r, carry):
        for c in range(D // 16):
            rows_v[0, r, pl.ds(c * 16, 16)] = zero16
        return carry
    lax.fori_loop(0, K, _z, 0)

    def _zs(r, carry):
        for c in range(128 // 16):
            s_loc[r, pl.ds(c * 16, 16)] = zero16
        return carry
    lax.fori_loop(0, AL_R, _zs, 0)

    def _za(t, carry):
        pltpu.async_copy(rows_v.at[0],
                         acc_sp.at[pl.ds(sid * ROWS_PER_TILE + t * K, K)], gsem)
        return carry
    lax.fori_loop(0, ROWS_PER_TILE // K, _za, 0)

    def _zw(t, carry):
        pltpu.make_async_copy(
            rows_v.at[0],
            acc_sp.at[pl.ds(sid * ROWS_PER_TILE + t * K, K)], gsem).wait()
        return carry
    lax.fori_loop(0, ROWS_PER_TILE // K, _zw, 0)
    plsc.subcore_barrier()

    def _sblock(sb, carry):
        base = tile_base + sb * SB
        pltpu.sync_copy(src_hbm.at[pl.ds(base, SB)], src_sb)
        pltpu.sync_copy(dst_hbm.at[pl.ds(base, SB)], dst_sb)

        pltpu.async_copy(hs_hbm.at[src_sb.at[0]], rows_v.at[0], gsem)

        def _chunk(j, c1):
            b = j % 2
            @pl.when(j >= 1)
            def _():
                pltpu.make_async_copy(rows_v.at[1 - b],
                                      acc_sp.at[dst_sb.at[j - 1]],
                                      ssem).wait()

            @pl.when(j < SB - 1)
            def _():
                pltpu.async_copy(hs_hbm.at[src_sb.at[j + 1]],
                                 rows_v.at[1 - b], gsem)

            def _ee(t, c2):
                sv = src_sb[j, pl.ds(t * 16, 16)]
                dv = dst_sb[j, pl.ds(t * 16, 16)]
                e = (plsc.load_gather(as_v, [sv >> 7, sv & 127])
                     + plsc.load_gather(ad_v, [dv >> 7, dv & 127]))
                e = jnp.where(e > 0.0, e, 0.2 * e)
                ee = jnp.exp(e)
                ee_v[t] = ee
                plsc.addupdate_scatter(s_loc, [dv >> 7, dv & 127], ee)
                return c2
            lax.fori_loop(0, K // 16, _ee, 0)

            pltpu.make_async_copy(hs_hbm.at[src_sb.at[j]],
                                  rows_v.at[b], gsem).wait()

            def _row(i):
                w = plsc.load_gather(
                    ee_v, [jnp.broadcast_to(i >> 4, (16,)),
                           jnp.broadcast_to(i & 15, (16,))])
                for c in range(D // 16):
                    rows_v[b, i, pl.ds(c * 16, 16)] = (
                        rows_v[b, i, pl.ds(c * 16, 16)] * w)
            plsc.parallel_loop(0, K, 1, unroll=4)(_row)

            pltpu.async_copy(rows_v.at[b], acc_sp.at[dst_sb.at[j]],
                             ssem, add=True)
            return c1
        lax.fori_loop(0, SB, _chunk, 0)

        pltpu.make_async_copy(rows_v.at[(SB - 1) % 2],
                              acc_sp.at[dst_sb.at[SB - 1]], ssem).wait()
        return carry
    lax.fori_loop(0, nsb, _sblock, 0)

    pltpu.sync_copy(s_loc, s_hbm.at[sid * 2 + cid])
    plsc.subcore_barrier()

    def _wb(t, carry):
        pltpu.sync_copy(
            acc_sp.at[pl.ds(sid * ROWS_PER_TILE + t * ZR, ZR)],
            out_hbm.at[cid, pl.ds(sid * ROWS_PER_TILE + t * ZR, ZR)])
        return carry
    lax.fori_loop(0, ROWS_PER_TILE // ZR, _wb, 0)


_sc_edge = functools.partial(
    pl.kernel,
    out_type=(jax.ShapeDtypeStruct((2, NP, D), jnp.float32),
              jax.ShapeDtypeStruct((NW, AL_R, 128), jnp.float32)),
    mesh=plsc.VectorSubcoreMesh(core_axis_name="c", subcore_axis_name="s"),
    scratch_types=[
        pltpu.VMEM((SB, K), jnp.int32),
        pltpu.VMEM((SB, K), jnp.int32),
        pltpu.VMEM((AL_R, 128), jnp.float32),
        pltpu.VMEM((AL_R, 128), jnp.float32),
        pltpu.VMEM((K // 16, 16), jnp.float32),
        pltpu.VMEM((2, K, D), jnp.float32),
        pltpu.VMEM((AL_R, 128), jnp.float32),
        pltpu.VMEM_SHARED((NP, D), jnp.float32),
        pltpu.SemaphoreType.DMA,
        pltpu.SemaphoreType.DMA,
    ],
    compiler_params=pltpu.CompilerParams(needs_layout_passes=False),
    )(_sc_edge_body)


def kernel(x, edge_index, W1s, W1d, a1s, a1d, b1, W2, a2s, a2d, b2,
           W3, a3s, a3d, b3, lin1_W, lin1_b, lin2_W, lin2_b):
    f32 = jnp.float32
    x_p = jnp.zeros((NP, D), f32).at[:N_NODES].set(x)
    pad = jnp.full((EP - N_EDGES,), PAD_NODE, jnp.int32)
    src = jnp.concatenate([edge_index[0], pad]).reshape(EP // K, K)
    dst = jnp.concatenate([edge_index[1], pad]).reshape(EP // K, K)

    def col(a):
        return a.reshape(D, 1)

    def row(a, w=D):
        return a.reshape(1, w)

    def tr(sv):
        svt = sv.reshape(NW, AL_R * 128).transpose(1, 0)
        return jnp.zeros((NP, NW), jnp.float32).at[:AL_R * 128].set(svt)

    hs, als, ald = _tc_first(x_p, W1s, W1d, col(a1s), col(a1d))
    acc, sv = _sc_edge(hs, src, dst, als, ald)
    hs, als, ald = _tc_mid(acc, tr(sv), row(b1), W2, col(a2s), col(a2d))
    acc, sv = _sc_edge(hs, src, dst, als, ald)
    hs, als, ald = _tc_mid(acc, tr(sv), row(b2), W3, col(a3s), col(a3d))
    acc, sv = _sc_edge(hs, src, dst, als, ald)

    w2p = jnp.zeros((D, D), f32).at[:, :D_OUT].set(lin2_W)
    b2p = jnp.zeros((D,), f32).at[:D_OUT].set(lin2_b)
    out = _tc_last(acc, tr(sv), row(b3), lin1_W, row(lin1_b), w2p, row(b2p))
    return out[:N_NODES, :D_OUT]


# speedup vs baseline: 1.5408x; 1.0999x over previous
import functools

import jax
import jax.numpy as jnp
from jax import lax
from jax.experimental import pallas as pl
from jax.experimental.pallas import tpu as pltpu
from jax.experimental.pallas import tpu_sc as plsc

N_NODES = 10000
N_EDGES = 320000
D = 128
D_OUT = 64

NP = 10240
EP = 327680
PAD_NODE = 10100

NW = 32
EDGES_PER_TILE = EP // NW
K = 64
SB = 2
CH0 = 208
CH1 = 112
AL_R = 80
CHUNKS = EDGES_PER_TILE // K
ROWS_PER_TILE = NP // 16
ZR = 128

_R = 2048
_G = NP // _R
_AR = _R // D


def _tc_first_body(x_ref, ws_ref, wd_ref, as_ref, ad_ref, hs_ref, als_ref, ald_ref):
    x = x_ref[...]
    hs = jnp.dot(x, ws_ref[...], preferred_element_type=jnp.float32)
    hs_ref[...] = hs
    als = jnp.dot(hs, as_ref[...], preferred_element_type=jnp.float32)
    als_ref[...] = als.reshape(_AR, D)
    v = jnp.dot(wd_ref[...], ad_ref[...], preferred_element_type=jnp.float32)
    ald_ref[...] = jnp.dot(x, v, preferred_element_type=jnp.float32).reshape(_AR, D)


def _tc_mid_body(acc_ref, s_ref, b_ref, w_ref, as_ref, ad_ref, hs_ref, als_ref, ald_ref):
    num = acc_ref[0] + acc_ref[1]
    s = jnp.sum(s_ref[...], axis=1, keepdims=True)
    h = jnp.maximum(jnp.where(s > 0.0, num / s, 0.0) + b_ref[...], 0.0)
    hs = jnp.dot(h, w_ref[...], preferred_element_type=jnp.float32)
    hs_ref[...] = hs
    als = jnp.dot(hs, as_ref[...], preferred_element_type=jnp.float32)
    als_ref[...] = als.reshape(_AR, D)
    v = jnp.dot(w_ref[...], ad_ref[...], preferred_element_type=jnp.float32)
    ald_ref[...] = jnp.dot(h, v, preferred_element_type=jnp.float32).reshape(_AR, D)


def _tc_last_body(acc_ref, s_ref, b_ref, w1_ref, b1_ref, w2_ref, b2_ref, out_ref):
    num = acc_ref[0] + acc_ref[1]
    s = jnp.sum(s_ref[...], axis=1, keepdims=True)
    h = jnp.maximum(jnp.where(s > 0.0, num / s, 0.0) + b_ref[...], 0.0)
    h = jnp.maximum(jnp.dot(h, w1_ref[...], preferred_element_type=jnp.float32)
                    + b1_ref[...], 0.0)
    out_ref[...] = jnp.dot(h, w2_ref[...], preferred_element_type=jnp.float32) + b2_ref[...]


def _row_blk(i):
    return (i, 0)


def _acc_blk(i):
    return (0, i, 0)


def _full_blk(i):
    return (0, 0)


_W_SPEC = pl.BlockSpec((D, D), _full_blk)
_A_SPEC = pl.BlockSpec((D, 1), _full_blk)
_B_SPEC = pl.BlockSpec((1, D), _full_blk)
_H_SPEC = pl.BlockSpec((_R, D), _row_blk)
_AL_SPEC = pl.BlockSpec((_AR, D), _row_blk)
_ACC_SPEC = pl.BlockSpec((2, _R, D), _acc_blk)
_S_SPEC = pl.BlockSpec((_R, NW), _row_blk)

_PROJ_OUT = (jax.ShapeDtypeStruct((NP, D), jnp.float32),
             jax.ShapeDtypeStruct((NP // D, D), jnp.float32),
             jax.ShapeDtypeStruct((NP // D, D), jnp.float32))

_tc_first = pl.pallas_call(
    _tc_first_body, grid=(_G,),
    in_specs=[_H_SPEC, _W_SPEC, _W_SPEC, _A_SPEC, _A_SPEC],
    out_specs=[_H_SPEC, _AL_SPEC, _AL_SPEC],
    out_shape=_PROJ_OUT)

_tc_mid = pl.pallas_call(
    _tc_mid_body, grid=(_G,),
    in_specs=[_ACC_SPEC, _S_SPEC, _B_SPEC, _W_SPEC, _A_SPEC, _A_SPEC],
    out_specs=[_H_SPEC, _AL_SPEC, _AL_SPEC],
    out_shape=_PROJ_OUT)

_tc_last = pl.pallas_call(
    _tc_last_body, grid=(_G,),
    in_specs=[_ACC_SPEC, _S_SPEC, _B_SPEC, _W_SPEC, _B_SPEC, _W_SPEC, _B_SPEC],
    out_specs=_H_SPEC,
    out_shape=jax.ShapeDtypeStruct((NP, D), jnp.float32))


def _sc_edge_body(hs_hbm, src_hbm, dst_hbm, as_hbm, ad_hbm, out_hbm, s_hbm,
                  src_sb, dst_sb, as_v, ad_v, ee_v, rows_v, s_loc, acc_sp,
                  gsem, ssem, isem):
    cid = lax.axis_index("c")
    sid = lax.axis_index("s")
    tile_base = sid * (CH0 + CH1) + cid * CH0
    nsb = jnp.where(cid == 0, CH0 // SB, CH1 // SB)

    pltpu.sync_copy(as_hbm.at[pl.ds(0, AL_R)], as_v)
    pltpu.sync_copy(ad_hbm.at[pl.ds(0, AL_R)], ad_v)

    zero16 = jnp.zeros((16,), jnp.float32)

    def _z(r, carry):
        for c in range(D // 16):
            rows_v[0, r, pl.ds(c * 16, 16)] = zero16
        return carry
    lax.fori_loop(0, K, _z, 0)

    def _zs(r, carry):
        for c in range(128 // 16):
            s_loc[r, pl.ds(c * 16, 16)] = zero16
        return carry
    lax.fori_loop(0, AL_R, _zs, 0)

    def _za(t, carry):
        pltpu.async_copy(rows_v.at[0],
                         acc_sp.at[pl.ds(sid * ROWS_PER_TILE + t * K, K)], gsem)
        return carry
    lax.fori_loop(0, ROWS_PER_TILE // K, _za, 0)

    def _zw(t, carry):
        pltpu.make_async_copy(
            rows_v.at[0],
            acc_sp.at[pl.ds(sid * ROWS_PER_TILE + t * K, K)], gsem).wait()
        return carry
    lax.fori_loop(0, ROWS_PER_TILE // K, _zw, 0)
    plsc.subcore_barrier()

    def _stage_idx(sb, islot):
        base = tile_base + sb * SB
        pltpu.async_copy(src_hbm.at[pl.ds(base, SB)], src_sb.at[islot], isem)
        pltpu.async_copy(dst_hbm.at[pl.ds(base, SB)], dst_sb.at[islot], isem)

    def _wait_idx():
        pltpu.make_async_copy(src_hbm.at[pl.ds(0, SB)], src_sb.at[0],
                              isem).wait()
        pltpu.make_async_copy(dst_hbm.at[pl.ds(0, SB)], dst_sb.at[0],
                              isem).wait()

    def _wait_gather(b):
        pltpu.make_async_copy(hs_hbm.at[src_sb.at[0, 0]], rows_v.at[b],
                              gsem).wait()

    def _wait_scatter(b):
        pltpu.make_async_copy(rows_v.at[b], acc_sp.at[dst_sb.at[0, 0]],
                              ssem).wait()

    nch = jnp.where(cid == 0, CH0, CH1)
    nsb = nch // SB

    _stage_idx(0, 0)
    _wait_idx()
    pltpu.async_copy(hs_hbm.at[src_sb.at[0, 0]], rows_v.at[0], gsem)

    def _chunk(gj, carry):
        b = gj % 2
        sb = gj // SB
        jj = gj % SB
        islot = sb % 2

        @pl.when(gj >= 1)
        def _():
            _wait_scatter(1 - b)

        @pl.when((jj == 0) & (sb + 1 <= nsb - 1))
        def _():
            _stage_idx(sb + 1, 1 - islot)

        def _ee(t, c2):
            sv = src_sb[islot, jj, pl.ds(t * 16, 16)]
            dv = dst_sb[islot, jj, pl.ds(t * 16, 16)]
            e = (plsc.load_gather(as_v, [sv >> 7, sv & 127])
                 + plsc.load_gather(ad_v, [dv >> 7, dv & 127]))
            e = jnp.where(e > 0.0, e, 0.2 * e)
            ee = jnp.exp(e)
            ee_v[t] = ee
            plsc.addupdate_scatter(s_loc, [dv >> 7, dv & 127], ee)
            return c2
        lax.fori_loop(0, K // 16, _ee, 0)

        @pl.when((jj == SB - 1) & (gj + 1 < nch))
        def _():
            _wait_idx()

        @pl.when(gj + 1 < nch)
        def _():
            islot1 = ((gj + 1) // SB) % 2
            jj1 = (gj + 1) % SB
            pltpu.async_copy(hs_hbm.at[src_sb.at[islot1, jj1]],
                             rows_v.at[1 - b], gsem)

        _wait_gather(b)

        def _row(i):
            w = plsc.load_gather(
                ee_v, [jnp.broadcast_to(i >> 4, (16,)),
                       jnp.broadcast_to(i & 15, (16,))])
            for c in range(D // 16):
                rows_v[b, i, pl.ds(c * 16, 16)] = (
                    rows_v[b, i, pl.ds(c * 16, 16)] * w)
        plsc.parallel_loop(0, K, 1, unroll=4)(_row)

        pltpu.async_copy(rows_v.at[b], acc_sp.at[dst_sb.at[islot, jj]],
                         ssem, add=True)
        return carry
    lax.fori_loop(0, nch, _chunk, 0)
    _wait_scatter((nch - 1) % 2)

    pltpu.sync_copy(s_loc, s_hbm.at[sid * 2 + cid])
    plsc.subcore_barrier()

    def _wb(t, carry):
        pltpu.sync_copy(
            acc_sp.at[pl.ds(sid * ROWS_PER_TILE + t * ZR, ZR)],
            out_hbm.at[cid, pl.ds(sid * ROWS_PER_TILE + t * ZR, ZR)])
        return carry
    lax.fori_loop(0, ROWS_PER_TILE // ZR, _wb, 0)


_sc_edge = functools.partial(
    pl.kernel,
    out_type=(jax.ShapeDtypeStruct((2, NP, D), jnp.float32),
              jax.ShapeDtypeStruct((NW, AL_R, 128), jnp.float32)),
    mesh=plsc.VectorSubcoreMesh(core_axis_name="c", subcore_axis_name="s"),
    scratch_types=[
        pltpu.VMEM((2, SB, K), jnp.int32),
        pltpu.VMEM((2, SB, K), jnp.int32),
        pltpu.VMEM((AL_R, 128), jnp.float32),
        pltpu.VMEM((AL_R, 128), jnp.float32),
        pltpu.VMEM((K // 16, 16), jnp.float32),
        pltpu.VMEM((2, K, D), jnp.float32),
        pltpu.VMEM((AL_R, 128), jnp.float32),
        pltpu.VMEM_SHARED((NP, D), jnp.float32),
        pltpu.SemaphoreType.DMA,
        pltpu.SemaphoreType.DMA,
        pltpu.SemaphoreType.DMA,
    ],
    compiler_params=pltpu.CompilerParams(needs_layout_passes=False),
    )(_sc_edge_body)


def kernel(x, edge_index, W1s, W1d, a1s, a1d, b1, W2, a2s, a2d, b2,
           W3, a3s, a3d, b3, lin1_W, lin1_b, lin2_W, lin2_b):
    f32 = jnp.float32
    x_p = jnp.zeros((NP, D), f32).at[:N_NODES].set(x)
    pad = jnp.full((EP - N_EDGES,), PAD_NODE, jnp.int32)
    src = jnp.concatenate([edge_index[0], pad]).reshape(EP // K, K)
    dst = jnp.concatenate([edge_index[1], pad]).reshape(EP // K, K)

    def col(a):
        return a.reshape(D, 1)

    def row(a, w=D):
        return a.reshape(1, w)

    def tr(sv):
        svt = sv.reshape(NW, AL_R * 128).transpose(1, 0)
        return jnp.zeros((NP, NW), jnp.float32).at[:AL_R * 128].set(svt)

    hs, als, ald = _tc_first(x_p, W1s, W1d, col(a1s), col(a1d))
    acc, sv = _sc_edge(hs, src, dst, als, ald)
    hs, als, ald = _tc_mid(acc, tr(sv), row(b1), W2, col(a2s), col(a2d))
    acc, sv = _sc_edge(hs, src, dst, als, ald)
    hs, als, ald = _tc_mid(acc, tr(sv), row(b2), W3, col(a3s), col(a3d))
    acc, sv = _sc_edge(hs, src, dst, als, ald)

    w2p = jnp.zeros((D, D), f32).at[:, :D_OUT].set(lin2_W)
    b2p = jnp.zeros((D,), f32).at[:D_OUT].set(lin2_b)
    out = _tc_last(acc, tr(sv), row(b3), lin1_W, row(lin1_b), w2p, row(b2p))
    return out[:N_NODES, :D_OUT]


# speedup vs baseline: 1.5608x; 1.0130x over previous
import functools

import jax
import jax.numpy as jnp
from jax import lax
from jax.experimental import pallas as pl
from jax.experimental.pallas import tpu as pltpu
from jax.experimental.pallas import tpu_sc as plsc

N_NODES = 10000
N_EDGES = 320000
D = 128
D_OUT = 64

NP = 10240
EP = 327680
PAD_NODE = 10100

NW = 32
EDGES_PER_TILE = EP // NW
K = 64
SB = 2
CH0 = 246
CH1 = 74
AL_R = 80
CHUNKS = EDGES_PER_TILE // K
ROWS_PER_TILE = NP // 16
ZR = 128

_R = 2048
_G = NP // _R
_AR = _R // D


def _tc_first_body(x_ref, ws_ref, wd_ref, as_ref, ad_ref, hs_ref, als_ref, ald_ref):
    x = x_ref[...]
    hs = jnp.dot(x, ws_ref[...], preferred_element_type=jnp.float32)
    hs_ref[...] = hs
    als = jnp.dot(hs, as_ref[...], preferred_element_type=jnp.float32)
    als_ref[...] = als.reshape(_AR, D)
    v = jnp.dot(wd_ref[...], ad_ref[...], preferred_element_type=jnp.float32)
    ald_ref[...] = jnp.dot(x, v, preferred_element_type=jnp.float32).reshape(_AR, D)


def _tc_mid_body(acc_ref, s_ref, b_ref, w_ref, as_ref, ad_ref, hs_ref, als_ref, ald_ref):
    num = acc_ref[0] + acc_ref[1]
    s = jnp.sum(s_ref[...], axis=1, keepdims=True)
    h = jnp.maximum(jnp.where(s > 0.0, num / s, 0.0) + b_ref[...], 0.0)
    hs = jnp.dot(h, w_ref[...], preferred_element_type=jnp.float32)
    hs_ref[...] = hs
    als = jnp.dot(hs, as_ref[...], preferred_element_type=jnp.float32)
    als_ref[...] = als.reshape(_AR, D)
    v = jnp.dot(w_ref[...], ad_ref[...], preferred_element_type=jnp.float32)
    ald_ref[...] = jnp.dot(h, v, preferred_element_type=jnp.float32).reshape(_AR, D)


def _tc_last_body(acc_ref, s_ref, b_ref, w1_ref, b1_ref, w2_ref, b2_ref, out_ref):
    num = acc_ref[0] + acc_ref[1]
    s = jnp.sum(s_ref[...], axis=1, keepdims=True)
    h = jnp.maximum(jnp.where(s > 0.0, num / s, 0.0) + b_ref[...], 0.0)
    h = jnp.maximum(jnp.dot(h, w1_ref[...], preferred_element_type=jnp.float32)
                    + b1_ref[...], 0.0)
    out_ref[...] = jnp.dot(h, w2_ref[...], preferred_element_type=jnp.float32) + b2_ref[...]


def _row_blk(i):
    return (i, 0)


def _acc_blk(i):
    return (0, i, 0)


def _full_blk(i):
    return (0, 0)


_W_SPEC = pl.BlockSpec((D, D), _full_blk)
_A_SPEC = pl.BlockSpec((D, 1), _full_blk)
_B_SPEC = pl.BlockSpec((1, D), _full_blk)
_H_SPEC = pl.BlockSpec((_R, D), _row_blk)
_AL_SPEC = pl.BlockSpec((_AR, D), _row_blk)
_ACC_SPEC = pl.BlockSpec((2, _R, D), _acc_blk)
_S_SPEC = pl.BlockSpec((_R, NW), _row_blk)

_PROJ_OUT = (jax.ShapeDtypeStruct((NP, D), jnp.float32),
             jax.ShapeDtypeStruct((NP // D, D), jnp.float32),
             jax.ShapeDtypeStruct((NP // D, D), jnp.float32))

_tc_first = pl.pallas_call(
    _tc_first_body, grid=(_G,),
    in_specs=[_H_SPEC, _W_SPEC, _W_SPEC, _A_SPEC, _A_SPEC],
    out_specs=[_H_SPEC, _AL_SPEC, _AL_SPEC],
    out_shape=_PROJ_OUT)

_tc_mid = pl.pallas_call(
    _tc_mid_body, grid=(_G,),
    in_specs=[_ACC_SPEC, _S_SPEC, _B_SPEC, _W_SPEC, _A_SPEC, _A_SPEC],
    out_specs=[_H_SPEC, _AL_SPEC, _AL_SPEC],
    out_shape=_PROJ_OUT)

_tc_last = pl.pallas_call(
    _tc_last_body, grid=(_G,),
    in_specs=[_ACC_SPEC, _S_SPEC, _B_SPEC, _W_SPEC, _B_SPEC, _W_SPEC, _B_SPEC],
    out_specs=_H_SPEC,
    out_shape=jax.ShapeDtypeStruct((NP, D), jnp.float32))


def _sc_edge_body(hs_hbm, src_hbm, dst_hbm, as_hbm, ad_hbm, out_hbm, s_hbm,
                  src_sb, dst_sb, as_v, ad_v, ee_v, rows_v, s_loc, acc_sp,
                  gsem, ssem, isem):
    cid = lax.axis_index("c")
    sid = lax.axis_index("s")
    tile_base = sid * (CH0 + CH1) + cid * CH0
    nsb = jnp.where(cid == 0, CH0 // SB, CH1 // SB)

    pltpu.sync_copy(as_hbm.at[pl.ds(0, AL_R)], as_v)
    pltpu.sync_copy(ad_hbm.at[pl.ds(0, AL_R)], ad_v)

    zero16 = jnp.zeros((16,), jnp.float32)

    def _z(r, carry):
        for c in range(D // 16):
            rows_v[0, r, pl.ds(c * 16, 16)] = zero16
        return carry
    lax.fori_loop(0, K, _z, 0)

    def _zs(r, carry):
        for c in range(128 // 16):
            s_loc[r, pl.ds(c * 16, 16)] = zero16
        return carry
    lax.fori_loop(0, AL_R, _zs, 0)

    def _za(t, carry):
        pltpu.async_copy(rows_v.at[0],
                         acc_sp.at[pl.ds(sid * ROWS_PER_TILE + t * K, K)], gsem)
        return carry
    lax.fori_loop(0, ROWS_PER_TILE // K, _za, 0)

    def _zw(t, carry):
        pltpu.make_async_copy(
            rows_v.at[0],
            acc_sp.at[pl.ds(sid * ROWS_PER_TILE + t * K, K)], gsem).wait()
        return carry
    lax.fori_loop(0, ROWS_PER_TILE // K, _zw, 0)
    plsc.subcore_barrier()

    def _stage_idx(sb, islot):
        base = tile_base + sb * SB
        pltpu.async_copy(src_hbm.at[pl.ds(base, SB)], src_sb.at[islot], isem)
        pltpu.async_copy(dst_hbm.at[pl.ds(base, SB)], dst_sb.at[islot], isem)

    def _wait_idx():
        pltpu.make_async_copy(src_hbm.at[pl.ds(0, SB)], src_sb.at[0],
                              isem).wait()
        pltpu.make_async_copy(dst_hbm.at[pl.ds(0, SB)], dst_sb.at[0],
                              isem).wait()

    def _wait_gather(b):
        pltpu.make_async_copy(hs_hbm.at[src_sb.at[0, 0]], rows_v.at[b],
                              gsem).wait()

    def _wait_scatter(b):
        pltpu.make_async_copy(rows_v.at[b], acc_sp.at[dst_sb.at[0, 0]],
                              ssem).wait()

    nch = jnp.where(cid == 0, CH0, CH1)
    nsb = nch // SB

    _stage_idx(0, 0)
    _wait_idx()
    pltpu.async_copy(hs_hbm.at[src_sb.at[0, 0]], rows_v.at[0], gsem)

    def _chunk(gj, carry):
        b = gj % 2
        sb = gj // SB
        jj = gj % SB
        islot = sb % 2

        @pl.when(gj >= 1)
        def _():
            _wait_scatter(1 - b)

        @pl.when((jj == 0) & (sb + 1 <= nsb - 1))
        def _():
            _stage_idx(sb + 1, 1 - islot)

        def _ee(t, c2):
            sv = src_sb[islot, jj, pl.ds(t * 16, 16)]
            dv = dst_sb[islot, jj, pl.ds(t * 16, 16)]
            e = (plsc.load_gather(as_v, [sv >> 7, sv & 127])
                 + plsc.load_gather(ad_v, [dv >> 7, dv & 127]))
            e = jnp.where(e > 0.0, e, 0.2 * e)
            ee = jnp.exp(e)
            ee_v[t] = ee
            plsc.addupdate_scatter(s_loc, [dv >> 7, dv & 127], ee)
            return c2
        lax.fori_loop(0, K // 16, _ee, 0)

        @pl.when((jj == SB - 1) & (gj + 1 < nch))
        def _():
            _wait_idx()

        @pl.when(gj + 1 < nch)
        def _():
            islot1 = ((gj + 1) // SB) % 2
            jj1 = (gj + 1) % SB
            pltpu.async_copy(hs_hbm.at[src_sb.at[islot1, jj1]],
                             rows_v.at[1 - b], gsem)

        _wait_gather(b)

        def _row(i):
            w = plsc.load_gather(
                ee_v, [jnp.broadcast_to(i >> 4, (16,)),
                       jnp.broadcast_to(i & 15, (16,))])
            for c in range(D // 16):
                rows_v[b, i, pl.ds(c * 16, 16)] = (
                    rows_v[b, i, pl.ds(c * 16, 16)] * w)
        plsc.parallel_loop(0, K, 1, unroll=4)(_row)

        pltpu.async_copy(rows_v.at[b], acc_sp.at[dst_sb.at[islot, jj]],
                         ssem, add=True)
        return carry
    lax.fori_loop(0, nch, _chunk, 0)
    _wait_scatter((nch - 1) % 2)

    pltpu.sync_copy(s_loc, s_hbm.at[sid * 2 + cid])
    plsc.subcore_barrier()

    def _wb(t, carry):
        pltpu.sync_copy(
            acc_sp.at[pl.ds(sid * ROWS_PER_TILE + t * ZR, ZR)],
            out_hbm.at[cid, pl.ds(sid * ROWS_PER_TILE + t * ZR, ZR)])
        return carry
    lax.fori_loop(0, ROWS_PER_TILE // ZR, _wb, 0)


_sc_edge = functools.partial(
    pl.kernel,
    out_type=(jax.ShapeDtypeStruct((2, NP, D), jnp.float32),
              jax.ShapeDtypeStruct((NW, AL_R, 128), jnp.float32)),
    mesh=plsc.VectorSubcoreMesh(core_axis_name="c", subcore_axis_name="s"),
    scratch_types=[
        pltpu.VMEM((2, SB, K), jnp.int32),
        pltpu.VMEM((2, SB, K), jnp.int32),
        pltpu.VMEM((AL_R, 128), jnp.float32),
        pltpu.VMEM((AL_R, 128), jnp.float32),
        pltpu.VMEM((K // 16, 16), jnp.float32),
        pltpu.VMEM((2, K, D), jnp.float32),
        pltpu.VMEM((AL_R, 128), jnp.float32),
        pltpu.VMEM_SHARED((NP, D), jnp.float32),
        pltpu.SemaphoreType.DMA,
        pltpu.SemaphoreType.DMA,
        pltpu.SemaphoreType.DMA,
    ],
    compiler_params=pltpu.CompilerParams(needs_layout_passes=False),
    )(_sc_edge_body)


def kernel(x, edge_index, W1s, W1d, a1s, a1d, b1, W2, a2s, a2d, b2,
           W3, a3s, a3d, b3, lin1_W, lin1_b, lin2_W, lin2_b):
    f32 = jnp.float32
    x_p = jnp.zeros((NP, D), f32).at[:N_NODES].set(x)
    pad = jnp.full((EP - N_EDGES,), PAD_NODE, jnp.int32)
    src = jnp.concatenate([edge_index[0], pad]).reshape(EP // K, K)
    dst = jnp.concatenate([edge_index[1], pad]).reshape(EP // K, K)

    def col(a):
        return a.reshape(D, 1)

    def row(a, w=D):
        return a.reshape(1, w)

    def tr(sv):
        svt = sv.reshape(NW, AL_R * 128).transpose(1, 0)
        return jnp.zeros((NP, NW), jnp.float32).at[:AL_R * 128].set(svt)

    hs, als, ald = _tc_first(x_p, W1s, W1d, col(a1s), col(a1d))
    acc, sv = _sc_edge(hs, src, dst, als, ald)
    hs, als, ald = _tc_mid(acc, tr(sv), row(b1), W2, col(a2s), col(a2d))
    acc, sv = _sc_edge(hs, src, dst, als, ald)
    hs, als, ald = _tc_mid(acc, tr(sv), row(b2), W3, col(a3s), col(a3d))
    acc, sv = _sc_edge(hs, src, dst, als, ald)

    w2p = jnp.zeros((D, D), f32).at[:, :D_OUT].set(lin2_W)
    b2p = jnp.zeros((D,), f32).at[:D_OUT].set(lin2_b)
    out = _tc_last(acc, tr(sv), row(b3), lin1_W, row(lin1_b), w2p, row(b2p))
    return out[:N_NODES, :D_OUT]
